# Initial kernel scaffold; baseline (speedup 1.0000x reference)
import jax
import jax.numpy as jnp
from jax.experimental import pallas as pl


def kernel(img):
    raise NotImplementedError("write your pallas kernel here")


# speedup vs baseline: 10.8844x; 10.8844x over previous
import functools

import jax
import jax.numpy as jnp
from jax import lax
from jax.experimental import pallas as pl
from jax.experimental.pallas import tpu as pltpu
from jax.experimental.pallas import tpu_sc as plsc

C = 192
R = 512
W = 512
B = C * R
NC = 2
NS = 16
NW = NC * NS
BPW = B // NW
CHUNK = 96
NCHUNKS = BPW // CHUNK

_MESH = plsc.VectorSubcoreMesh(core_axis_name="c", subcore_axis_name="s")


@functools.partial(
    pl.kernel,
    out_type=jax.ShapeDtypeStruct((B, W), jnp.float32),
    mesh=_MESH,
    scratch_types=[
        pltpu.VMEM((BPW,), jnp.int32),
        pltpu.VMEM((CHUNK, W), jnp.float32),
        pltpu.VMEM((CHUNK, W), jnp.float32),
        pltpu.SemaphoreType.DMA,
        pltpu.SemaphoreType.DMA,
        pltpu.SemaphoreType.DMA,
        pltpu.SemaphoreType.DMA,
    ],
)
def _sc_row_gather(img_hbm, idx_hbm, out_hbm, idx_v, buf0, buf1,
                   gsem0, gsem1, ssem0, ssem1):
    wid = lax.axis_index("s") * NC + lax.axis_index("c")
    base = wid * BPW

    pltpu.sync_copy(idx_hbm.at[pl.ds(base, BPW)], idx_v)

    bufs = (buf0, buf1)
    gsems = (gsem0, gsem1)
    ssems = (ssem0, ssem1)

    def gather_copy(c, b):
        off = pl.multiple_of(c * CHUNK, CHUNK)
        return pltpu.make_async_copy(img_hbm.at[idx_v.at[pl.ds(off, CHUNK)]],
                                     bufs[b], gsems[b])

    def scatter_copy(c, b):
        row0 = base + pl.multiple_of(c * CHUNK, CHUNK)
        return pltpu.make_async_copy(bufs[b], out_hbm.at[pl.ds(row0, CHUNK)],
                                     ssems[b])

    gather_copy(0, 0).start()
    gather_copy(1, 1).start()

    @pl.loop(0, NCHUNKS // 2)
    def _pair(i):
        for b in range(2):
            c = i * 2 + b
            gather_copy(c, b).wait()
            sc = scatter_copy(c, b)
            sc.start()
            sc.wait()

            @pl.when(c + 2 < NCHUNKS)
            def _():
                gather_copy(c + 2, b).start()


def kernel(img):
    rows = img.shape[1]
    src = jnp.arange(0, rows)
    noise = jax.random.normal(jax.random.key(42), (rows,), dtype=jnp.float32)
    dst = jnp.clip(jnp.round(noise + src.astype(jnp.float32)),
                   0, rows - 1).astype(jnp.int32)
    idx_flat = (jnp.arange(C, dtype=jnp.int32)[:, None] * R
                + dst[None, :]).reshape(B)
    out_flat = _sc_row_gather(img.reshape(B, W), idx_flat)
    return out_flat.reshape(C, R, W)
